# baseline (speedup 1.0000x reference)
import functools

import jax
import jax.numpy as jnp
from jax.experimental import pallas as pl
from jax.experimental.pallas import tpu as pltpu

_SMOOTHING = 0.1
_CONFIDENCE = 1.0 - _SMOOTHING
_IGNORE_INDEX = -100
_LANES = 128


def _loss_body(nblocks, num_classes, block_c,
               pred_ref, tgt_ref, out_ref, m_ref, s_ref, sx_ref, g_ref):
    j = pl.program_id(0)
    n = pred_ref.shape[0]
    nch = block_c // _LANES
    lane = jax.lax.broadcasted_iota(jnp.int32, (1, _LANES), 1)
    tgt = tgt_ref[...]

    @pl.when(j == 0)
    def _init():
        m_ref[...] = jnp.full((n, _LANES), -jnp.inf, jnp.float32)
        s_ref[...] = jnp.zeros((n, _LANES), jnp.float32)
        sx_ref[...] = jnp.zeros((n, _LANES), jnp.float32)
        g_ref[...] = jnp.zeros((n, _LANES), jnp.float32)

    x = pred_ref[...]

    def chunk_update(masked):
        chunks = []
        valids = []
        bmax = None
        for k in range(nch):
            xc = x[:, k * _LANES:(k + 1) * _LANES]
            if masked:
                cols = j * block_c + k * _LANES + lane
                v = cols < num_classes
                valids.append(v)
                xc = jnp.where(v, xc, -jnp.inf)
            chunks.append(xc)
            bmax = xc if bmax is None else jnp.maximum(bmax, xc)
        m_prev = m_ref[...]
        m_new = jnp.maximum(m_prev, bmax)
        ssum = s_ref[...] * jnp.exp(m_prev - m_new)
        sx = sx_ref[...]
        g = g_ref[...]
        for k, xc in enumerate(chunks):
            ssum = ssum + jnp.exp(xc - m_new)
            if masked:
                sx = sx + jnp.where(valids[k], xc, 0.0)
            else:
                sx = sx + xc
            rel = tgt - (j * block_c + k * _LANES)
            g = g + jnp.where(rel == lane, xc, 0.0)
        m_ref[...] = m_new
        s_ref[...] = ssum
        sx_ref[...] = sx
        g_ref[...] = g

    @pl.when(j < nblocks - 1)
    def _fast():
        chunk_update(masked=False)

    @pl.when(j == nblocks - 1)
    def _last():
        chunk_update(masked=True)
        m_l = m_ref[...]
        m_row = jnp.max(m_l, axis=1, keepdims=True)
        s_row = jnp.sum(s_ref[...] * jnp.exp(m_l - m_row),
                        axis=1, keepdims=True)
        lse = m_row + jnp.log(s_row)
        sumx_row = jnp.sum(sx_ref[...], axis=1, keepdims=True)
        g_row = jnp.sum(g_ref[...], axis=1, keepdims=True)
        sum_logp = sumx_row - num_classes * lse
        logp_t = g_row - lse
        eps = _SMOOTHING / (num_classes - 1)
        row_loss = -eps * sum_logp - (_CONFIDENCE - eps) * logp_t
        maskf = (tgt != _IGNORE_INDEX).astype(jnp.float32)
        loss = jnp.sum(row_loss * maskf) / jnp.sum(maskf)
        out_ref[...] = loss.reshape(1, 1)


def kernel(pred, target):
    n, num_classes = pred.shape
    block_c = 2048
    nblocks = pl.cdiv(num_classes, block_c)
    tgt2 = target.reshape(n, 1)

    out = pl.pallas_call(
        functools.partial(_loss_body, nblocks, num_classes, block_c),
        grid=(nblocks,),
        in_specs=[
            pl.BlockSpec((n, block_c), lambda j: (0, j)),
            pl.BlockSpec((n, 1), lambda j: (0, 0)),
        ],
        out_specs=pl.BlockSpec((1, 1), lambda j: (0, 0)),
        out_shape=jax.ShapeDtypeStruct((1, 1), jnp.float32),
        scratch_shapes=[pltpu.VMEM((n, _LANES), jnp.float32)] * 4,
    )(pred, tgt2)
    return out[0, 0]


# speedup vs baseline: 1.0944x; 1.0944x over previous
import functools

import jax
import jax.numpy as jnp
from jax.experimental import pallas as pl
from jax.experimental.pallas import tpu as pltpu

_SMOOTHING = 0.1
_CONFIDENCE = 1.0 - _SMOOTHING
_IGNORE_INDEX = -100
_LANES = 128


def _loss_body(nblocks, num_classes, block_c,
               pred_ref, tgt_ref, out_ref,
               m_ref, s_ref, sx_ref, g_ref, d_ref):
    j = pl.program_id(0)
    n = pred_ref.shape[0]
    nch = block_c // _LANES
    lane = jax.lax.broadcasted_iota(jnp.int32, (1, _LANES), 1)
    tgt = tgt_ref[...]

    @pl.when(j == 0)
    def _init():
        m_ref[...] = jnp.full((n, _LANES), -jnp.inf, jnp.float32)
        s_ref[...] = jnp.zeros((n, _LANES), jnp.float32)
        sx_ref[...] = jnp.zeros((n, _LANES), jnp.float32)
        g_ref[...] = jnp.zeros((n, _LANES), jnp.float32)
        d_ref[...] = tgt - lane

    def block_update(masked):
        bmax = None
        for k in range(nch):
            xc = pred_ref[:, k * _LANES:(k + 1) * _LANES]
            if masked:
                v = (j * block_c + k * _LANES + lane) < num_classes
                xc = jnp.where(v, xc, -jnp.inf)
            bmax = xc if bmax is None else jnp.maximum(bmax, xc)
        m_prev = m_ref[...]
        m_new = jnp.maximum(m_prev, jnp.max(bmax, axis=1, keepdims=True))
        alpha = jnp.exp(m_prev - m_new)
        ssum = s_ref[...] * alpha
        sx = sx_ref[...]
        g = g_ref[...]
        d = d_ref[...]
        for k in range(nch):
            xc = pred_ref[:, k * _LANES:(k + 1) * _LANES]
            if masked:
                v = (j * block_c + k * _LANES + lane) < num_classes
                ssum = ssum + jnp.exp(jnp.where(v, xc, -jnp.inf) - m_new)
                sx = sx + jnp.where(v, xc, 0.0)
            else:
                ssum = ssum + jnp.exp(xc - m_new)
                sx = sx + xc
            g = g + jnp.where(d == j * block_c + k * _LANES, xc, 0.0)
        m_ref[...] = m_new
        s_ref[...] = ssum
        sx_ref[...] = sx
        g_ref[...] = g

    @pl.when(j < nblocks - 1)
    def _fast():
        block_update(masked=False)

    @pl.when(j == nblocks - 1)
    def _last():
        block_update(masked=True)
        m_l = m_ref[...]
        m_row = jnp.max(m_l, axis=1, keepdims=True)
        s_row = jnp.sum(s_ref[...] * jnp.exp(m_l - m_row),
                        axis=1, keepdims=True)
        lse = m_row + jnp.log(s_row)
        sumx_row = jnp.sum(sx_ref[...], axis=1, keepdims=True)
        g_row = jnp.sum(g_ref[...], axis=1, keepdims=True)
        sum_logp = sumx_row - num_classes * lse
        logp_t = g_row - lse
        eps = _SMOOTHING / (num_classes - 1)
        row_loss = -eps * sum_logp - (_CONFIDENCE - eps) * logp_t
        maskf = (tgt != _IGNORE_INDEX).astype(jnp.float32)
        loss = jnp.sum(row_loss * maskf) / jnp.sum(maskf)
        out_ref[...] = loss.reshape(1, 1)


def kernel(pred, target):
    n, num_classes = pred.shape
    block_c = 4096
    nblocks = pl.cdiv(num_classes, block_c)
    tgt2 = target.reshape(n, 1)

    out = pl.pallas_call(
        functools.partial(_loss_body, nblocks, num_classes, block_c),
        grid=(nblocks,),
        in_specs=[
            pl.BlockSpec((n, block_c), lambda j: (0, j)),
            pl.BlockSpec((n, 1), lambda j: (0, 0)),
        ],
        out_specs=pl.BlockSpec((1, 1), lambda j: (0, 0)),
        out_shape=jax.ShapeDtypeStruct((1, 1), jnp.float32),
        scratch_shapes=[pltpu.VMEM((n, _LANES), jnp.float32)] * 4
        + [pltpu.VMEM((n, _LANES), jnp.int32)],
    )(pred, tgt2)
    return out[0, 0]
